# baseline (speedup 1.0000x reference)
import functools

import jax
import jax.numpy as jnp
from jax import lax
from jax.experimental import pallas as pl
from jax.experimental.pallas import tpu as pltpu
from jax.experimental.pallas import tpu_sc as plsc

B = 16384
DIM = 32
LANES = 16

_info = plsc.get_sparse_core_info()
_NC, _NS = _info.num_cores, _info.num_subcores
_NW = _NC * _NS
_BPW = B // _NW
_NCHUNK = 4
_CHUNK = _BPW // _NCHUNK


def _mf_body(u_hbm, i_hbm, U_hbm, V_hbm, bu_hbm, bi_hbm, mu_hbm, out_hbm,
             u_idx, i_idx, u_rows, v_rows, bu_rows, bi_rows, mu_v, out_v, sem):
    wid = lax.axis_index("s") * _NC + lax.axis_index("c")
    base = wid * _BPW

    for c in range(_NCHUNK):
        pltpu.sync_copy(u_hbm.at[pl.ds(base + c * _CHUNK, _CHUNK)], u_idx.at[c])
        pltpu.sync_copy(i_hbm.at[pl.ds(base + c * _CHUNK, _CHUNK)], i_idx.at[c])
    pltpu.sync_copy(mu_hbm, mu_v.at[pl.ds(0, 1)])

    cps = []
    for c in range(_NCHUNK):
        sl = pl.ds(c * _CHUNK, _CHUNK)
        cps.append(pltpu.async_copy(U_hbm.at[u_idx.at[c]], u_rows.at[sl], sem))
        cps.append(pltpu.async_copy(V_hbm.at[i_idx.at[c]], v_rows.at[sl], sem))
        cps.append(pltpu.async_copy(bu_hbm.at[u_idx.at[c]], bu_rows.at[c], sem))
        cps.append(pltpu.async_copy(bi_hbm.at[i_idx.at[c]], bi_rows.at[c], sem))
    for cp in cps:
        cp.wait()

    mu_s = mu_v[...][0]
    lane = lax.iota(jnp.int32, LANES)

    for g in range(_BPW // LANES):
        c = g // (_CHUNK // LANES)
        r0 = (g % (_CHUNK // LANES)) * LANES
        rows = g * LANES + lane
        acc0 = (bu_rows.at[c][pl.ds(r0, LANES)]
                + bi_rows.at[c][pl.ds(r0, LANES)] + mu_s)

        def body(d, acc, rows=rows):
            dvec = jnp.full((LANES,), 0, jnp.int32) + d
            return acc + (plsc.load_gather(u_rows, [rows, dvec])
                          * plsc.load_gather(v_rows, [rows, dvec]))

        acc = lax.fori_loop(0, DIM, body, acc0)
        out_v[pl.ds(g * LANES, LANES)] = acc

    pltpu.sync_copy(out_v, out_hbm.at[pl.ds(base, _BPW)])


@functools.partial(jax.jit, static_argnums=())
def _mf_sc(u, i, U, V, bu, bi, mu):
    mesh = plsc.VectorSubcoreMesh(core_axis_name="c", subcore_axis_name="s")
    return pl.kernel(
        _mf_body,
        mesh=mesh,
        compiler_params=pltpu.CompilerParams(
            use_tc_tiling_on_sc=False, needs_layout_passes=False),
        out_type=jax.ShapeDtypeStruct((B,), jnp.float32),
        scratch_types=[
            pltpu.VMEM((_NCHUNK, _CHUNK), jnp.int32),
            pltpu.VMEM((_NCHUNK, _CHUNK), jnp.int32),
            pltpu.VMEM((_BPW, DIM), jnp.float32),
            pltpu.VMEM((_BPW, DIM), jnp.float32),
            pltpu.VMEM((_NCHUNK, _CHUNK), jnp.float32),
            pltpu.VMEM((_NCHUNK, _CHUNK), jnp.float32),
            pltpu.VMEM((LANES,), jnp.float32),
            pltpu.VMEM((_BPW,), jnp.float32),
            pltpu.SemaphoreType.DMA,
        ],
    )(u, i, U, V, bu, bi, mu)


def kernel(u, i, U, V, bu, bi, mu):
    return _mf_sc(u, i, U, V, bu.reshape(-1), bi.reshape(-1), mu)
